# baseline (speedup 1.0000x reference)
import functools

import jax
import jax.numpy as jnp
from jax import lax
from jax.experimental import pallas as pl
from jax.experimental.pallas import tpu as pltpu
from jax.experimental.pallas import tpu_sc as plsc

N = 10000
E = 160000
D = 256
NW = 32
NP = 10240
NPW = NP // NW
TRASH = NPW

C0 = 8000
NCHUNK = E // C0
ECAP = 168256
G = 48
RB = 48
REFILL = RB * G

_mesh = plsc.VectorSubcoreMesh(core_axis_name="c", subcore_axis_name="s")


def _wid():
    return lax.axis_index("c") * 16 + lax.axis_index("s")


@functools.partial(
    pl.kernel,
    mesh=_mesh,
    out_type=[
        jax.ShapeDtypeStruct((NP,), jnp.float32),
        jax.ShapeDtypeStruct((NW * ECAP,), jnp.int32),
        jax.ShapeDtypeStruct((NW * ECAP,), jnp.int32),
        jax.ShapeDtypeStruct((NW * 16,), jnp.int32),
    ],
    scratch_types=[
        pltpu.VMEM((NPW + 16,), jnp.float32),
        pltpu.VMEM((C0,), jnp.int32),
        pltpu.VMEM((C0,), jnp.int32),
        pltpu.VMEM((C0 + 32,), jnp.int32),
        pltpu.VMEM((C0 + 32,), jnp.int32),
        pltpu.VMEM((16,), jnp.int32),
    ],
    compiler_params=pltpu.CompilerParams(needs_layout_passes=False),
)
def _prep(src_hbm, dst_hbm, cnt_hbm, csrc_hbm, cdl_hbm, nw_hbm,
          cnt_v, dbuf, sbuf, st_s, st_d, bounce):
    w = _wid()
    base = pl.multiple_of(w * NPW, 8)
    lbase = pl.multiple_of(w * ECAP, 8)

    for i in range((NPW + 16) // 16):
        cnt_v[pl.ds(i * 16, 16)] = jnp.zeros((16,), jnp.float32)

    zeros16 = jnp.zeros((16,), jnp.int32)
    trash16 = jnp.full((16,), TRASH, jnp.int32)
    ones16 = jnp.full((16,), 1, jnp.int32)
    one_hot = (lax.iota(jnp.int32, 16) == zeros16).astype(jnp.float32)
    basev = jnp.full((16,), base, jnp.int32)
    topv = basev + jnp.full((16,), NPW, jnp.int32)

    def chunk_body(c, off):
        pltpu.sync_copy(dst_hbm.at[pl.ds(c * C0, C0)], dbuf)
        pltpu.sync_copy(src_hbm.at[pl.ds(c * C0, C0)], sbuf)

        def grp(j, o_vec):
            d = dbuf[pl.ds(j * 16, 16)]
            s = sbuf[pl.ds(j * 16, 16)]
            m = (d >= basev) & (d < topv)
            pos = o_vec + plsc.cumsum(m.astype(jnp.int32)) - ones16
            plsc.store_scatter(st_s, [pos], s, mask=m)
            plsc.store_scatter(st_d, [pos], d - basev, mask=m)
            return o_vec + plsc.all_reduce_population_count(m)

        o_vec = lax.fori_loop(0, C0 // 16, grp, zeros16, unroll=4)
        n = o_vec[0]

        def inc(i, _):
            dl = st_d[pl.ds(i, 16)][0]
            plsc.addupdate(cnt_v.at[pl.ds(dl, 16)], one_hot)
            return 0

        lax.fori_loop(0, n, inc, 0)

        st_s[pl.ds(n, 16)] = zeros16
        st_d[pl.ds(n, 16)] = trash16
        n8 = (n + 7) & ~7

        off8 = pl.multiple_of(off, 8)
        pltpu.sync_copy(st_s.at[pl.ds(0, C0 + 16)],
                        csrc_hbm.at[pl.ds(lbase + off8, C0 + 16)])
        pltpu.sync_copy(st_d.at[pl.ds(0, C0 + 16)],
                        cdl_hbm.at[pl.ds(lbase + off8, C0 + 16)])
        return off + n8

    off = lax.fori_loop(0, NCHUNK, chunk_body, 0)

    for i in range(7):
        st_s[pl.ds(i * 16, 16)] = zeros16
        st_d[pl.ds(i * 16, 16)] = trash16
    off8 = pl.multiple_of(off, 8)
    pltpu.sync_copy(st_s.at[pl.ds(0, 112)],
                    csrc_hbm.at[pl.ds(lbase + off8, 112)])
    pltpu.sync_copy(st_d.at[pl.ds(0, 112)],
                    cdl_hbm.at[pl.ds(lbase + off8, 112)])

    pltpu.sync_copy(cnt_v.at[pl.ds(0, NPW)], cnt_hbm.at[pl.ds(base, NPW)])
    bounce[...] = jnp.full((16,), off, jnp.int32)
    pltpu.sync_copy(bounce, nw_hbm.at[pl.ds(pl.multiple_of(w * 16, 8), 16)])


@functools.partial(
    pl.kernel,
    mesh=_mesh,
    out_type=jax.ShapeDtypeStruct((NP, D), jnp.float32),
    scratch_types=[
        pltpu.VMEM((NPW + 1, D), jnp.float32),
        pltpu.VMEM((REFILL,), jnp.int32),
        pltpu.VMEM((REFILL + 16,), jnp.int32),
        pltpu.VMEM((G, D), jnp.float32),
        pltpu.VMEM((G, D), jnp.float32),
        pltpu.VMEM((G, D), jnp.float32),
        pltpu.VMEM((16,), jnp.int32),
        pltpu.SemaphoreType.DMA,
        pltpu.SemaphoreType.DMA,
        pltpu.SemaphoreType.DMA,
    ],
    compiler_params=pltpu.CompilerParams(needs_layout_passes=False),
)
def _agg(g_hbm, csrc_hbm, cdl_hbm, nw_hbm, acc_hbm,
         accum, sbuf, dbuf, stag0, stag1, stag2, bounce, sem0, sem1, sem2):
    w = _wid()
    base = pl.multiple_of(w * NPW, 8)
    lbase = pl.multiple_of(w * ECAP, 8)

    pltpu.sync_copy(nw_hbm.at[pl.ds(pl.multiple_of(w * 16, 8), 16)], bounce)
    n = bounce[...][0]
    pltpu.sync_copy(g_hbm.at[pl.ds(base, NPW)], accum.at[pl.ds(0, NPW)])

    nbat = (n + (G - 1)) // G
    slots = ((stag0, sem0), (stag1, sem1), (stag2, sem2))

    def gather(b, stag, sem):
        pltpu.make_async_copy(
            g_hbm.at[sbuf.at[pl.ds(b * G, G)]], stag, sem).start()

    def wait(stag, sem):
        pltpu.make_async_copy(g_hbm.at[pl.ds(0, G)], stag, sem).wait()

    def accrow(stag, dl_base):
        def row(i, _):
            dl = dbuf[pl.ds(dl_base + i, 16)][0]
            for c in range(D // 16):
                sl = pl.ds(c * 16, 16)
                accum[dl, sl] = accum[dl, sl] + stag[i, sl]
            return 0
        lax.fori_loop(0, G, row, 0)

    def refill_body(r, _):
        roff = pl.multiple_of(lbase + r * REFILL, 8)
        pltpu.sync_copy(csrc_hbm.at[pl.ds(roff, REFILL)], sbuf)
        pltpu.sync_copy(cdl_hbm.at[pl.ds(roff, REFILL)],
                        dbuf.at[pl.ds(0, REFILL)])
        nb = jnp.minimum(RB, nbat - r * RB)

        gather(0, stag0, sem0)
        for j in (1, 2):
            @pl.when(j < nb)
            def _(j=j):
                gather(j, *slots[j])

        def trio(kk, _):
            k3 = kk * 3
            for j in range(3):
                stag, sem = slots[j]

                @pl.when(k3 + j < nb)
                def _(stag=stag, sem=sem, j=j):
                    wait(stag, sem)
                    accrow(stag, (k3 + j) * G)

                    @pl.when(k3 + j + 3 < nb)
                    def _(stag=stag, sem=sem, j=j):
                        gather(k3 + j + 3, stag, sem)

            return 0

        lax.fori_loop(0, (nb + 2) // 3, trio, 0)
        return 0

    lax.fori_loop(0, (nbat + (RB - 1)) // RB, refill_body, 0)

    pltpu.sync_copy(accum.at[pl.ds(0, NPW)], acc_hbm.at[pl.ds(base, NPW)])


_RB_TC = 1280
_GRID = NP // _RB_TC


def _mm1_body(x_ref, w_ref, cnt_ref, o_ref):
    dinv = lax.rsqrt(cnt_ref[...] + 1.0)
    o_ref[...] = jnp.dot(x_ref[...] * dinv, w_ref[...],
                         preferred_element_type=jnp.float32)


def _mm2_body(a_ref, w_ref, b_ref, cnt_ref, o_ref):
    dinv = lax.rsqrt(cnt_ref[...] + 1.0)
    h = a_ref[...] * dinv + b_ref[...]
    h = jnp.where(h > 0, h, 0.1 * h)
    o_ref[...] = jnp.dot(h * dinv, w_ref[...],
                         preferred_element_type=jnp.float32)


def _fin_body(a_ref, b_ref, cnt_ref, o_ref):
    dinv = lax.rsqrt(cnt_ref[...] + 1.0)
    o_ref[...] = a_ref[...] * dinv + b_ref[...]


_row_spec = pl.BlockSpec((_RB_TC, D), lambda i: (i, 0))
_w_spec = pl.BlockSpec((D, D), lambda i: (0, 0))
_b_spec = pl.BlockSpec((1, D), lambda i: (0, 0))
_cnt_spec = pl.BlockSpec((_RB_TC, 1), lambda i: (i, 0))
_out_f32 = jax.ShapeDtypeStruct((NP, D), jnp.float32)

_mm1 = pl.pallas_call(
    _mm1_body, grid=(_GRID,),
    in_specs=[_row_spec, _w_spec, _cnt_spec],
    out_specs=_row_spec, out_shape=_out_f32)

_mm2 = pl.pallas_call(
    _mm2_body, grid=(_GRID,),
    in_specs=[_row_spec, _w_spec, _b_spec, _cnt_spec],
    out_specs=_row_spec, out_shape=_out_f32)

_fin = pl.pallas_call(
    _fin_body, grid=(_GRID,),
    in_specs=[_row_spec, _b_spec, _cnt_spec],
    out_specs=_row_spec, out_shape=_out_f32)


def kernel(x, edge_index, W1, b1, W2, b2, W3, b3):
    src = edge_index[0].astype(jnp.int32)
    dst = edge_index[1].astype(jnp.int32)
    x_pad = jnp.zeros((NP, D), jnp.float32).at[:N].set(x)

    cnt, csrc, cdl, nw = _prep(src, dst)
    cnt2 = cnt.reshape(NP, 1)

    g = _mm1(x_pad, W1, cnt2)
    a = _agg(g, csrc, cdl, nw)
    g = _mm2(a, W2, b1.reshape(1, D), cnt2)
    a = _agg(g, csrc, cdl, nw)
    g = _mm2(a, W3, b2.reshape(1, D), cnt2)
    a = _agg(g, csrc, cdl, nw)
    out = _fin(a, b3.reshape(1, D), cnt2)
    return out[:N]


# speedup vs baseline: 1.9584x; 1.9584x over previous
import functools

import jax
import jax.numpy as jnp
from jax import lax
from jax.experimental import pallas as pl
from jax.experimental.pallas import tpu as pltpu
from jax.experimental.pallas import tpu_sc as plsc

N = 10000
E = 160000
D = 256
NW = 32
NP = 10240
NPW = NP // NW
TRASH = NPW

C0 = 8000
NCHUNK = E // C0
ECAP = 168256
G = 48
RB = 48
REFILL = RB * G

_mesh = plsc.VectorSubcoreMesh(core_axis_name="c", subcore_axis_name="s")


def _wid():
    return lax.axis_index("c") * 16 + lax.axis_index("s")


@functools.partial(
    pl.kernel,
    mesh=_mesh,
    out_type=[
        jax.ShapeDtypeStruct((NP,), jnp.float32),
        jax.ShapeDtypeStruct((NW * ECAP,), jnp.int32),
        jax.ShapeDtypeStruct((NW * ECAP,), jnp.int32),
        jax.ShapeDtypeStruct((NW * 16,), jnp.int32),
    ],
    scratch_types=[
        pltpu.VMEM((NPW + 16,), jnp.float32),
        pltpu.VMEM((C0,), jnp.int32),
        pltpu.VMEM((C0,), jnp.int32),
        pltpu.VMEM((C0 + 32,), jnp.int32),
        pltpu.VMEM((C0 + 32,), jnp.int32),
        pltpu.VMEM((16,), jnp.int32),
    ],
    compiler_params=pltpu.CompilerParams(needs_layout_passes=False),
)
def _prep(src_hbm, dst_hbm, cnt_hbm, csrc_hbm, cdl_hbm, nw_hbm,
          cnt_v, dbuf, sbuf, st_s, st_d, bounce):
    w = _wid()
    base = pl.multiple_of(w * NPW, 8)
    lbase = pl.multiple_of(w * ECAP, 8)

    for i in range((NPW + 16) // 16):
        cnt_v[pl.ds(i * 16, 16)] = jnp.zeros((16,), jnp.float32)

    zeros16 = jnp.zeros((16,), jnp.int32)
    trash16 = jnp.full((16,), TRASH, jnp.int32)
    ones16 = jnp.full((16,), 1, jnp.int32)
    one_hot = (lax.iota(jnp.int32, 16) == zeros16).astype(jnp.float32)
    basev = jnp.full((16,), base, jnp.int32)
    topv = basev + jnp.full((16,), NPW, jnp.int32)

    def chunk_body(c, off):
        pltpu.sync_copy(dst_hbm.at[pl.ds(c * C0, C0)], dbuf)
        pltpu.sync_copy(src_hbm.at[pl.ds(c * C0, C0)], sbuf)

        def grp(j, o_vec):
            d = dbuf[pl.ds(j * 16, 16)]
            s = sbuf[pl.ds(j * 16, 16)]
            m = (d >= basev) & (d < topv)
            pos = o_vec + plsc.cumsum(m.astype(jnp.int32)) - ones16
            plsc.store_scatter(st_s, [pos], s, mask=m)
            plsc.store_scatter(st_d, [pos], d - basev, mask=m)
            return o_vec + plsc.all_reduce_population_count(m)

        o_vec = lax.fori_loop(0, C0 // 16, grp, zeros16, unroll=4)
        n = o_vec[0]

        def inc(i, _):
            dl = st_d[pl.ds(i, 16)][0]
            plsc.addupdate(cnt_v.at[pl.ds(dl, 16)], one_hot)
            return 0

        lax.fori_loop(0, n, inc, 0)

        st_s[pl.ds(n, 16)] = zeros16
        st_d[pl.ds(n, 16)] = trash16
        n8 = (n + 7) & ~7

        off8 = pl.multiple_of(off, 8)
        pltpu.sync_copy(st_s.at[pl.ds(0, C0 + 16)],
                        csrc_hbm.at[pl.ds(lbase + off8, C0 + 16)])
        pltpu.sync_copy(st_d.at[pl.ds(0, C0 + 16)],
                        cdl_hbm.at[pl.ds(lbase + off8, C0 + 16)])
        return off + n8

    off = lax.fori_loop(0, NCHUNK, chunk_body, 0)

    for i in range(7):
        st_s[pl.ds(i * 16, 16)] = zeros16
        st_d[pl.ds(i * 16, 16)] = trash16
    off8 = pl.multiple_of(off, 8)
    pltpu.sync_copy(st_s.at[pl.ds(0, 112)],
                    csrc_hbm.at[pl.ds(lbase + off8, 112)])
    pltpu.sync_copy(st_d.at[pl.ds(0, 112)],
                    cdl_hbm.at[pl.ds(lbase + off8, 112)])

    pltpu.sync_copy(cnt_v.at[pl.ds(0, NPW)], cnt_hbm.at[pl.ds(base, NPW)])
    bounce[...] = jnp.full((16,), off, jnp.int32)
    pltpu.sync_copy(bounce, nw_hbm.at[pl.ds(pl.multiple_of(w * 16, 8), 16)])


@functools.partial(
    pl.kernel,
    mesh=_mesh,
    out_type=jax.ShapeDtypeStruct((NP, D), jnp.float32),
    scratch_types=[
        pltpu.VMEM((NPW + 1, D), jnp.float32),
        pltpu.VMEM((REFILL,), jnp.int32),
        pltpu.VMEM((REFILL + 16,), jnp.int32),
        pltpu.VMEM((G, D), jnp.float32),
        pltpu.VMEM((G, D), jnp.float32),
        pltpu.VMEM((G, D), jnp.float32),
        pltpu.VMEM((16,), jnp.int32),
        pltpu.SemaphoreType.DMA,
        pltpu.SemaphoreType.DMA,
        pltpu.SemaphoreType.DMA,
    ],
    compiler_params=pltpu.CompilerParams(needs_layout_passes=False),
)
def _agg(g_hbm, csrc_hbm, cdl_hbm, nw_hbm, acc_hbm,
         accum, sbuf, dbuf, stag0, stag1, stag2, bounce, sem0, sem1, sem2):
    w = _wid()
    base = pl.multiple_of(w * NPW, 8)
    lbase = pl.multiple_of(w * ECAP, 8)

    pltpu.sync_copy(nw_hbm.at[pl.ds(pl.multiple_of(w * 16, 8), 16)], bounce)
    n = bounce[...][0]
    pltpu.sync_copy(g_hbm.at[pl.ds(base, NPW)], accum.at[pl.ds(0, NPW)])

    nbat = (n + (G - 1)) // G
    slots = ((stag0, sem0), (stag1, sem1), (stag2, sem2))

    def gather(b, stag, sem):
        pltpu.make_async_copy(
            g_hbm.at[sbuf.at[pl.ds(b * G, G)]], stag, sem).start()

    def wait(stag, sem):
        pltpu.make_async_copy(g_hbm.at[pl.ds(0, G)], stag, sem).wait()

    def accrow(stag, dl_base):
        @plsc.parallel_loop(0, G, unroll=2)
        def _(i):
            dl = dbuf[pl.ds(dl_base + i, 16)][0]
            for c in range(D // 16):
                sl = pl.ds(c * 16, 16)
                plsc.addupdate(accum.at[dl, sl], stag[i, sl])

    def refill_body(r, _):
        roff = pl.multiple_of(lbase + r * REFILL, 8)
        pltpu.sync_copy(csrc_hbm.at[pl.ds(roff, REFILL)], sbuf)
        pltpu.sync_copy(cdl_hbm.at[pl.ds(roff, REFILL)],
                        dbuf.at[pl.ds(0, REFILL)])
        nb = jnp.minimum(RB, nbat - r * RB)

        gather(0, stag0, sem0)
        for j in (1, 2):
            @pl.when(j < nb)
            def _(j=j):
                gather(j, *slots[j])

        def trio(kk, _):
            k3 = kk * 3
            for j in range(3):
                stag, sem = slots[j]

                @pl.when(k3 + j < nb)
                def _(stag=stag, sem=sem, j=j):
                    wait(stag, sem)
                    accrow(stag, (k3 + j) * G)

                    @pl.when(k3 + j + 3 < nb)
                    def _(stag=stag, sem=sem, j=j):
                        gather(k3 + j + 3, stag, sem)

            return 0

        lax.fori_loop(0, (nb + 2) // 3, trio, 0)
        return 0

    lax.fori_loop(0, (nbat + (RB - 1)) // RB, refill_body, 0)

    pltpu.sync_copy(accum.at[pl.ds(0, NPW)], acc_hbm.at[pl.ds(base, NPW)])


_RB_TC = 1280
_GRID = NP // _RB_TC


def _mm1_body(x_ref, w_ref, cnt_ref, o_ref):
    dinv = lax.rsqrt(cnt_ref[...] + 1.0)
    o_ref[...] = jnp.dot(x_ref[...] * dinv, w_ref[...],
                         preferred_element_type=jnp.float32)


def _mm2_body(a_ref, w_ref, b_ref, cnt_ref, o_ref):
    dinv = lax.rsqrt(cnt_ref[...] + 1.0)
    h = a_ref[...] * dinv + b_ref[...]
    h = jnp.where(h > 0, h, 0.1 * h)
    o_ref[...] = jnp.dot(h * dinv, w_ref[...],
                         preferred_element_type=jnp.float32)


def _fin_body(a_ref, b_ref, cnt_ref, o_ref):
    dinv = lax.rsqrt(cnt_ref[...] + 1.0)
    o_ref[...] = a_ref[...] * dinv + b_ref[...]


_row_spec = pl.BlockSpec((_RB_TC, D), lambda i: (i, 0))
_w_spec = pl.BlockSpec((D, D), lambda i: (0, 0))
_b_spec = pl.BlockSpec((1, D), lambda i: (0, 0))
_cnt_spec = pl.BlockSpec((_RB_TC, 1), lambda i: (i, 0))
_out_f32 = jax.ShapeDtypeStruct((NP, D), jnp.float32)

_mm1 = pl.pallas_call(
    _mm1_body, grid=(_GRID,),
    in_specs=[_row_spec, _w_spec, _cnt_spec],
    out_specs=_row_spec, out_shape=_out_f32)

_mm2 = pl.pallas_call(
    _mm2_body, grid=(_GRID,),
    in_specs=[_row_spec, _w_spec, _b_spec, _cnt_spec],
    out_specs=_row_spec, out_shape=_out_f32)

_fin = pl.pallas_call(
    _fin_body, grid=(_GRID,),
    in_specs=[_row_spec, _b_spec, _cnt_spec],
    out_specs=_row_spec, out_shape=_out_f32)


def kernel(x, edge_index, W1, b1, W2, b2, W3, b3):
    src = edge_index[0].astype(jnp.int32)
    dst = edge_index[1].astype(jnp.int32)
    x_pad = jnp.zeros((NP, D), jnp.float32).at[:N].set(x)

    cnt, csrc, cdl, nw = _prep(src, dst)
    cnt2 = cnt.reshape(NP, 1)

    g = _mm1(x_pad, W1, cnt2)
    a = _agg(g, csrc, cdl, nw)
    g = _mm2(a, W2, b1.reshape(1, D), cnt2)
    a = _agg(g, csrc, cdl, nw)
    g = _mm2(a, W3, b2.reshape(1, D), cnt2)
    a = _agg(g, csrc, cdl, nw)
    out = _fin(a, b3.reshape(1, D), cnt2)
    return out[:N]


# speedup vs baseline: 2.3201x; 1.1847x over previous
import functools

import jax
import jax.numpy as jnp
from jax import lax
from jax.experimental import pallas as pl
from jax.experimental.pallas import tpu as pltpu
from jax.experimental.pallas import tpu_sc as plsc

N = 10000
E = 160000
D = 256
NW = 32
NP = 10240
NPW = NP // NW
TRASH = NPW

C0 = 8000
NCHUNK = E // C0
ECAP = 168256
G = 48
RB = 48
REFILL = RB * G

_mesh = plsc.VectorSubcoreMesh(core_axis_name="c", subcore_axis_name="s")


def _wid():
    return lax.axis_index("c") * 16 + lax.axis_index("s")


@functools.partial(
    pl.kernel,
    mesh=_mesh,
    out_type=[
        jax.ShapeDtypeStruct((NP,), jnp.float32),
        jax.ShapeDtypeStruct((NW * ECAP,), jnp.int32),
        jax.ShapeDtypeStruct((NW * ECAP,), jnp.int32),
        jax.ShapeDtypeStruct((NW * 16,), jnp.int32),
    ],
    scratch_types=[
        pltpu.VMEM((NPW + 16,), jnp.float32),
        pltpu.VMEM((C0,), jnp.int32),
        pltpu.VMEM((C0,), jnp.int32),
        pltpu.VMEM((C0 + 32,), jnp.int32),
        pltpu.VMEM((C0 + 32,), jnp.int32),
        pltpu.VMEM((16,), jnp.int32),
    ],
    compiler_params=pltpu.CompilerParams(needs_layout_passes=False),
)
def _prep(src_hbm, dst_hbm, cnt_hbm, csrc_hbm, cdl_hbm, nw_hbm,
          cnt_v, dbuf, sbuf, st_s, st_d, bounce):
    w = _wid()
    base = pl.multiple_of(w * NPW, 8)
    lbase = pl.multiple_of(w * ECAP, 8)

    for i in range((NPW + 16) // 16):
        cnt_v[pl.ds(i * 16, 16)] = jnp.zeros((16,), jnp.float32)

    zeros16 = jnp.zeros((16,), jnp.int32)
    trash16 = jnp.full((16,), TRASH, jnp.int32)
    ones16 = jnp.full((16,), 1, jnp.int32)
    one_hot = (lax.iota(jnp.int32, 16) == zeros16).astype(jnp.float32)
    basev = jnp.full((16,), base, jnp.int32)
    topv = basev + jnp.full((16,), NPW, jnp.int32)

    def chunk_body(c, off):
        pltpu.sync_copy(dst_hbm.at[pl.ds(c * C0, C0)], dbuf)
        pltpu.sync_copy(src_hbm.at[pl.ds(c * C0, C0)], sbuf)

        @plsc.parallel_loop(0, C0 // 16, unroll=2, carry=zeros16)
        def grp(j, o_vec):
            d = dbuf[pl.ds(j * 16, 16)]
            s = sbuf[pl.ds(j * 16, 16)]
            m = (d >= basev) & (d < topv)
            pos = o_vec + plsc.cumsum(m.astype(jnp.int32)) - ones16
            plsc.store_scatter(st_s, [pos], s, mask=m)
            plsc.store_scatter(st_d, [pos], d - basev, mask=m)
            return o_vec + plsc.all_reduce_population_count(m)

        n = grp[0]

        @plsc.parallel_loop(0, n, unroll=2)
        def _(i):
            dl = st_d[pl.ds(i, 16)][0]
            plsc.addupdate(cnt_v.at[pl.ds(dl, 16)], one_hot)

        st_s[pl.ds(n, 16)] = zeros16
        st_d[pl.ds(n, 16)] = trash16
        n8 = (n + 7) & ~7

        off8 = pl.multiple_of(off, 8)
        pltpu.sync_copy(st_s.at[pl.ds(0, C0 + 16)],
                        csrc_hbm.at[pl.ds(lbase + off8, C0 + 16)])
        pltpu.sync_copy(st_d.at[pl.ds(0, C0 + 16)],
                        cdl_hbm.at[pl.ds(lbase + off8, C0 + 16)])
        return off + n8

    off = lax.fori_loop(0, NCHUNK, chunk_body, 0)

    for i in range(7):
        st_s[pl.ds(i * 16, 16)] = zeros16
        st_d[pl.ds(i * 16, 16)] = trash16
    off8 = pl.multiple_of(off, 8)
    pltpu.sync_copy(st_s.at[pl.ds(0, 112)],
                    csrc_hbm.at[pl.ds(lbase + off8, 112)])
    pltpu.sync_copy(st_d.at[pl.ds(0, 112)],
                    cdl_hbm.at[pl.ds(lbase + off8, 112)])

    pltpu.sync_copy(cnt_v.at[pl.ds(0, NPW)], cnt_hbm.at[pl.ds(base, NPW)])
    bounce[...] = jnp.full((16,), off, jnp.int32)
    pltpu.sync_copy(bounce, nw_hbm.at[pl.ds(pl.multiple_of(w * 16, 8), 16)])


@functools.partial(
    pl.kernel,
    mesh=_mesh,
    out_type=jax.ShapeDtypeStruct((NP, D), jnp.float32),
    scratch_types=[
        pltpu.VMEM((NPW + 1, D), jnp.float32),
        pltpu.VMEM((REFILL,), jnp.int32),
        pltpu.VMEM((REFILL + 16,), jnp.int32),
        pltpu.VMEM((G, D), jnp.float32),
        pltpu.VMEM((G, D), jnp.float32),
        pltpu.VMEM((G, D), jnp.float32),
        pltpu.VMEM((16,), jnp.int32),
        pltpu.SemaphoreType.DMA,
        pltpu.SemaphoreType.DMA,
        pltpu.SemaphoreType.DMA,
    ],
    compiler_params=pltpu.CompilerParams(needs_layout_passes=False),
)
def _agg(g_hbm, csrc_hbm, cdl_hbm, nw_hbm, acc_hbm,
         accum, sbuf, dbuf, stag0, stag1, stag2, bounce, sem0, sem1, sem2):
    w = _wid()
    base = pl.multiple_of(w * NPW, 8)
    lbase = pl.multiple_of(w * ECAP, 8)

    pltpu.sync_copy(nw_hbm.at[pl.ds(pl.multiple_of(w * 16, 8), 16)], bounce)
    n = bounce[...][0]
    pltpu.sync_copy(g_hbm.at[pl.ds(base, NPW)], accum.at[pl.ds(0, NPW)])

    nbat = (n + (G - 1)) // G
    slots = ((stag0, sem0), (stag1, sem1), (stag2, sem2))

    def gather(b, stag, sem):
        pltpu.make_async_copy(
            g_hbm.at[sbuf.at[pl.ds(b * G, G)]], stag, sem).start()

    def wait(stag, sem):
        pltpu.make_async_copy(g_hbm.at[pl.ds(0, G)], stag, sem).wait()

    def accrow(stag, dl_base):
        @plsc.parallel_loop(0, G, unroll=2)
        def _(i):
            dl = dbuf[pl.ds(dl_base + i, 16)][0]
            for c in range(D // 16):
                sl = pl.ds(c * 16, 16)
                plsc.addupdate(accum.at[dl, sl], stag[i, sl])

    def refill_body(r, _):
        roff = pl.multiple_of(lbase + r * REFILL, 8)
        pltpu.sync_copy(csrc_hbm.at[pl.ds(roff, REFILL)], sbuf)
        pltpu.sync_copy(cdl_hbm.at[pl.ds(roff, REFILL)],
                        dbuf.at[pl.ds(0, REFILL)])
        nb = jnp.minimum(RB, nbat - r * RB)

        gather(0, stag0, sem0)
        for j in (1, 2):
            @pl.when(j < nb)
            def _(j=j):
                gather(j, *slots[j])

        def trio(kk, _):
            k3 = kk * 3
            for j in range(3):
                stag, sem = slots[j]

                @pl.when(k3 + j < nb)
                def _(stag=stag, sem=sem, j=j):
                    wait(stag, sem)
                    accrow(stag, (k3 + j) * G)

                    @pl.when(k3 + j + 3 < nb)
                    def _(stag=stag, sem=sem, j=j):
                        gather(k3 + j + 3, stag, sem)

            return 0

        lax.fori_loop(0, (nb + 2) // 3, trio, 0)
        return 0

    lax.fori_loop(0, (nbat + (RB - 1)) // RB, refill_body, 0)

    pltpu.sync_copy(accum.at[pl.ds(0, NPW)], acc_hbm.at[pl.ds(base, NPW)])


_RB_TC = 1280
_GRID = NP // _RB_TC


def _mm1_body(x_ref, w_ref, cnt_ref, o_ref):
    dinv = lax.rsqrt(cnt_ref[...] + 1.0)
    o_ref[...] = jnp.dot(x_ref[...] * dinv, w_ref[...],
                         preferred_element_type=jnp.float32)


def _mm2_body(a_ref, w_ref, b_ref, cnt_ref, o_ref):
    dinv = lax.rsqrt(cnt_ref[...] + 1.0)
    h = a_ref[...] * dinv + b_ref[...]
    h = jnp.where(h > 0, h, 0.1 * h)
    o_ref[...] = jnp.dot(h * dinv, w_ref[...],
                         preferred_element_type=jnp.float32)


def _fin_body(a_ref, b_ref, cnt_ref, o_ref):
    dinv = lax.rsqrt(cnt_ref[...] + 1.0)
    o_ref[...] = a_ref[...] * dinv + b_ref[...]


_row_spec = pl.BlockSpec((_RB_TC, D), lambda i: (i, 0))
_w_spec = pl.BlockSpec((D, D), lambda i: (0, 0))
_b_spec = pl.BlockSpec((1, D), lambda i: (0, 0))
_cnt_spec = pl.BlockSpec((_RB_TC, 1), lambda i: (i, 0))
_out_f32 = jax.ShapeDtypeStruct((NP, D), jnp.float32)

_mm1 = pl.pallas_call(
    _mm1_body, grid=(_GRID,),
    in_specs=[_row_spec, _w_spec, _cnt_spec],
    out_specs=_row_spec, out_shape=_out_f32)

_mm2 = pl.pallas_call(
    _mm2_body, grid=(_GRID,),
    in_specs=[_row_spec, _w_spec, _b_spec, _cnt_spec],
    out_specs=_row_spec, out_shape=_out_f32)

_fin = pl.pallas_call(
    _fin_body, grid=(_GRID,),
    in_specs=[_row_spec, _b_spec, _cnt_spec],
    out_specs=_row_spec, out_shape=_out_f32)


def kernel(x, edge_index, W1, b1, W2, b2, W3, b3):
    src = edge_index[0].astype(jnp.int32)
    dst = edge_index[1].astype(jnp.int32)
    x_pad = jnp.zeros((NP, D), jnp.float32).at[:N].set(x)

    cnt, csrc, cdl, nw = _prep(src, dst)
    cnt2 = cnt.reshape(NP, 1)

    g = _mm1(x_pad, W1, cnt2)
    a = _agg(g, csrc, cdl, nw)
    g = _mm2(a, W2, b1.reshape(1, D), cnt2)
    a = _agg(g, csrc, cdl, nw)
    g = _mm2(a, W3, b2.reshape(1, D), cnt2)
    a = _agg(g, csrc, cdl, nw)
    out = _fin(a, b3.reshape(1, D), cnt2)
    return out[:N]
